# baseline (speedup 1.0000x reference)
import functools

import jax
import jax.numpy as jnp
from jax import lax
from jax.experimental import pallas as pl
from jax.experimental.pallas import tpu as pltpu
from jax.experimental.pallas import tpu_sc as plsc

D = 512
R = 7
CH = 96
NB = 2
NC = 2
NS = 16
NW = NC * NS
L = 16


def _build_sc_kernel(n_total):
    b_per_w = n_total // NW
    n_ch = b_per_w // CH
    n_grp = n_ch // NB
    mesh = plsc.VectorSubcoreMesh(core_axis_name="c", subcore_axis_name="s")

    @functools.partial(
        pl.kernel,
        mesh=mesh,
        out_type=jax.ShapeDtypeStruct((n_total, D), jnp.float32),
        scratch_types=[
            pltpu.VMEM((b_per_w,), jnp.int32),
            pltpu.VMEM((b_per_w,), jnp.int32),
            pltpu.VMEM((NB, CH, D), jnp.float32),
            [pltpu.SemaphoreType.DMA] * NB,
            [pltpu.SemaphoreType.DMA] * NB,
        ],
    )
    def k(t_hbm, x0_hbm, x1_hbm, x2_hbm, x3_hbm, out_hbm,
          cidx, tmp, rows, gsems, ssems):
        wid = lax.axis_index("s") * NC + lax.axis_index("c")
        base = wid * b_per_w

        pltpu.sync_copy(x0_hbm.at[pl.ds(base, b_per_w)], cidx)

        def acc_pass(x_hbm):
            pltpu.sync_copy(x_hbm.at[pl.ds(base, b_per_w)], tmp)

            def body(i, _):
                sl = pl.ds(i * L, L)
                cidx[sl] = cidx[sl] * R + tmp[sl]
                return 0

            lax.fori_loop(0, b_per_w // L, body, 0)

        acc_pass(x1_hbm)
        acc_pass(x2_hbm)
        acc_pass(x3_hbm)

        def gather(c, b):
            idx = cidx.at[pl.ds(c * CH, CH)]
            pltpu.async_copy(t_hbm.at[idx], rows.at[b], gsems[b])

        def scatter(c, b):
            pltpu.async_copy(rows.at[b], out_hbm.at[pl.ds(base + c * CH, CH)],
                             ssems[b])

        def wait_g(b):
            pltpu.make_async_copy(out_hbm.at[pl.ds(base, CH)], rows.at[b],
                                  gsems[b]).wait()

        def wait_s(b):
            pltpu.make_async_copy(rows.at[b], out_hbm.at[pl.ds(base, CH)],
                                  ssems[b]).wait()

        for b in range(NB):
            gather(b, b)

        def grp(p, _):
            c0 = NB * p
            for b in range(NB):
                wait_g(b)
                scatter(c0 + b, b)

            @pl.when(p < n_grp - 1)
            def _():
                for b in range(NB):
                    wait_s(b)
                    gather(c0 + NB + b, b)

            return 0

        lax.fori_loop(0, n_grp, grp, 0)
        for b in range(NB):
            wait_s(b)

    return k


def kernel(x, hour_w, weekday_w, day_w, month_w):
    del month_w
    b, s, _ = x.shape
    n = b * s
    x = x.astype(jnp.int32)

    t = (day_w[:R][:, None, None, None, :]
         + day_w[:R][None, :, None, None, :]
         + weekday_w[:R][None, None, :, None, :]
         + hour_w[:R][None, None, None, :, :]).reshape(R ** 4, D)

    xf = x.reshape(n, 5)
    sc_k = _build_sc_kernel(n)
    out = sc_k(t, xf[:, 0].ravel(), xf[:, 1].ravel(),
               xf[:, 2].ravel(), xf[:, 3].ravel())
    return out.reshape(b, s, D)


# speedup vs baseline: 1.0033x; 1.0033x over previous
import functools

import jax
import jax.numpy as jnp
from jax import lax
from jax.experimental import pallas as pl
from jax.experimental.pallas import tpu as pltpu
from jax.experimental.pallas import tpu_sc as plsc

D = 512
R = 7
CH = 48
NB = 4
NC = 2
NS = 16
NW = NC * NS
L = 16


def _build_sc_kernel(n_total):
    b_per_w = n_total // NW
    n_ch = b_per_w // CH
    n_grp = n_ch // NB
    mesh = plsc.VectorSubcoreMesh(core_axis_name="c", subcore_axis_name="s")

    @functools.partial(
        pl.kernel,
        mesh=mesh,
        out_type=jax.ShapeDtypeStruct((n_total, D), jnp.float32),
        scratch_types=[
            pltpu.VMEM((b_per_w,), jnp.int32),
            pltpu.VMEM((b_per_w,), jnp.int32),
            pltpu.VMEM((NB, CH, D), jnp.float32),
            [pltpu.SemaphoreType.DMA] * NB,
            [pltpu.SemaphoreType.DMA] * NB,
        ],
    )
    def k(t_hbm, x0_hbm, x1_hbm, x2_hbm, x3_hbm, out_hbm,
          cidx, tmp, rows, gsems, ssems):
        wid = lax.axis_index("s") * NC + lax.axis_index("c")
        base = wid * b_per_w

        pltpu.sync_copy(x0_hbm.at[pl.ds(base, b_per_w)], cidx)

        def acc_pass(x_hbm):
            pltpu.sync_copy(x_hbm.at[pl.ds(base, b_per_w)], tmp)

            def body(i, _):
                sl = pl.ds(i * L, L)
                cidx[sl] = cidx[sl] * R + tmp[sl]
                return 0

            lax.fori_loop(0, b_per_w // L, body, 0)

        acc_pass(x1_hbm)
        acc_pass(x2_hbm)
        acc_pass(x3_hbm)

        def gather(c, b):
            idx = cidx.at[pl.ds(c * CH, CH)]
            pltpu.async_copy(t_hbm.at[idx], rows.at[b], gsems[b])

        def scatter(c, b):
            pltpu.async_copy(rows.at[b], out_hbm.at[pl.ds(base + c * CH, CH)],
                             ssems[b])

        def wait_g(b):
            pltpu.make_async_copy(out_hbm.at[pl.ds(base, CH)], rows.at[b],
                                  gsems[b]).wait()

        def wait_s(b):
            pltpu.make_async_copy(rows.at[b], out_hbm.at[pl.ds(base, CH)],
                                  ssems[b]).wait()

        for b in range(NB):
            gather(b, b)

        def grp(p, _):
            c0 = NB * p
            for b in range(NB):
                wait_g(b)
                scatter(c0 + b, b)

            @pl.when(p < n_grp - 1)
            def _():
                for b in range(NB):
                    wait_s(b)
                    gather(c0 + NB + b, b)

            return 0

        lax.fori_loop(0, n_grp, grp, 0)
        for b in range(NB):
            wait_s(b)

    return k


def kernel(x, hour_w, weekday_w, day_w, month_w):
    del month_w
    b, s, _ = x.shape
    n = b * s
    x = x.astype(jnp.int32)

    t = (day_w[:R][:, None, None, None, :]
         + day_w[:R][None, :, None, None, :]
         + weekday_w[:R][None, None, :, None, :]
         + hour_w[:R][None, None, None, :, :]).reshape(R ** 4, D)

    xf = x.reshape(n, 5)
    sc_k = _build_sc_kernel(n)
    out = sc_k(t, xf[:, 0].ravel(), xf[:, 1].ravel(),
               xf[:, 2].ravel(), xf[:, 3].ravel())
    return out.reshape(b, s, D)
